# baseline (speedup 1.0000x reference)
import functools

import jax
import jax.numpy as jnp
from jax import lax
from jax.experimental import pallas as pl
from jax.experimental.pallas import tpu as pltpu
from jax.experimental.pallas import tpu_sc as plsc

N = 10000
E = 320000
D = 128
H = 128
G = 64
NUM_LAYERS = 3

NC = 2
NS = 16
NW = NC * NS
CHK = 96
KPS = 15
NSUPER = 7
NCHUNKT = NSUPER * KPS
EPW = NCHUNKT * CHK
NPADR = 8
RPT = 640
RPT_LAST = N - (NS - 1) * RPT


def _sc_agg_body(h_hbm, src_hbm, dst_hbm, out_hbm, sidx0, sidx1, didx0, didx1,
                 rows0, rows1, rows2, acc,
                 gsem0, gsem1, gsem2, ssem0, ssem1, ssem2,
                 sisem0, sisem1, disem0, disem1):
    c = lax.axis_index("c")
    s = lax.axis_index("s")
    w = c * NS + s

    sidx = (sidx0, sidx1)
    didx = (didx0, didx1)
    rows = (rows0, rows1, rows2)
    gsems = (gsem0, gsem1, gsem2)
    ssems = (ssem0, ssem1, ssem2)
    sisems = (sisem0, sisem1)
    disems = (disem0, disem1)

    pltpu.sync_copy(src_hbm.at[w, 0], sidx0)
    pltpu.sync_copy(dst_hbm.at[w, 0], didx0)

    @pl.when(s < NS - 1)
    def _():
        pltpu.sync_copy(h_hbm.at[pl.ds(s * RPT, RPT)],
                        acc.at[pl.ds(s * RPT, RPT)])

    @pl.when(s == NS - 1)
    def _():
        pltpu.sync_copy(h_hbm.at[pl.ds((NS - 1) * RPT, RPT_LAST)],
                        acc.at[pl.ds((NS - 1) * RPT, RPT_LAST)])

    plsc.subcore_barrier()

    pltpu.async_copy(h_hbm.at[sidx0.at[0]], rows0, gsem0)
    pltpu.async_copy(h_hbm.at[sidx0.at[1]], rows1, gsem1)

    def super_block(j, sp):
        for k in range(KPS):
            i = j * KPS + k
            b = k % 3

            if k == 0:
                @pl.when(j > 0)
                def _():
                    pltpu.make_async_copy(dst_hbm.at[w, j], didx[sp],
                                          disems[sp]).wait()

            pltpu.make_async_copy(h_hbm.at[sidx[sp].at[0]], rows[b],
                                  gsems[b]).wait()
            pltpu.sync_copy(rows[b], acc.at[didx[sp].at[k]], add=True)

            if k == 1:
                @pl.when(j + 1 < NSUPER)
                def _():
                    pltpu.async_copy(src_hbm.at[w, j + 1], sidx[1 - sp],
                                     sisems[1 - sp])
                    pltpu.async_copy(dst_hbm.at[w, j + 1], didx[1 - sp],
                                     disems[1 - sp])

            if k == KPS - 2:
                @pl.when(j + 1 < NSUPER)
                def _():
                    pltpu.make_async_copy(src_hbm.at[w, j + 1], sidx[1 - sp],
                                          sisems[1 - sp]).wait()

            if k < KPS - 2:
                nidx = sidx[sp].at[k + 2]
            else:
                nidx = sidx[1 - sp].at[k - (KPS - 2)]

            @pl.when(i + 2 < NCHUNKT)
            def _():
                pltpu.async_copy(h_hbm.at[nidx], rows[(k + 2) % 3],
                                 gsems[(k + 2) % 3])

    @pl.loop(0, (NSUPER - 1) // 2)
    def _(jj):
        for sp in range(2):
            super_block(jj * 2 + sp, sp)

    super_block(NSUPER - 1, (NSUPER - 1) % 2)

    plsc.subcore_barrier()

    @pl.when(s < NS - 1)
    def _():
        pltpu.sync_copy(acc.at[pl.ds(s * RPT, RPT)],
                        out_hbm.at[c, pl.ds(s * RPT, RPT)])

    @pl.when(s == NS - 1)
    def _():
        pltpu.sync_copy(acc.at[pl.ds((NS - 1) * RPT, RPT_LAST)],
                        out_hbm.at[c, pl.ds((NS - 1) * RPT, RPT_LAST)])


@functools.cache
def _make_sc_agg():
    mesh = plsc.VectorSubcoreMesh(
        core_axis_name="c", subcore_axis_name="s", num_cores=NC, num_subcores=NS
    )
    return pl.kernel(
        _sc_agg_body,
        out_type=jax.ShapeDtypeStruct((NC, N, D), jnp.float32),
        mesh=mesh,
        scratch_types=[
            pltpu.VMEM((KPS, CHK), jnp.int32),
            pltpu.VMEM((KPS, CHK), jnp.int32),
            pltpu.VMEM((KPS, CHK), jnp.int32),
            pltpu.VMEM((KPS, CHK), jnp.int32),
            pltpu.VMEM((CHK, D), jnp.float32),
            pltpu.VMEM((CHK, D), jnp.float32),
            pltpu.VMEM((CHK, D), jnp.float32),
            pltpu.VMEM_SHARED((N, D), jnp.float32),
            pltpu.SemaphoreType.DMA,
            pltpu.SemaphoreType.DMA,
            pltpu.SemaphoreType.DMA,
            pltpu.SemaphoreType.DMA,
            pltpu.SemaphoreType.DMA,
            pltpu.SemaphoreType.DMA,
            pltpu.SemaphoreType.DMA,
            pltpu.SemaphoreType.DMA,
            pltpu.SemaphoreType.DMA,
            pltpu.SemaphoreType.DMA,
        ],
    )


def _mlp_body(p_ref, h_ref, epsm1_ref, w1t_ref, b1_ref, g1_ref, be1_ref,
              w2t_ref, b2_ref, g2_ref, be2_ref, o_ref):
    h = h_ref[0:N]
    z = p_ref[0] + p_ref[1] + epsm1_ref[...] * h
    z1 = jnp.dot(z, w1t_ref[...], preferred_element_type=jnp.float32) + b1_ref[...]
    mu = jnp.mean(z1, axis=0, keepdims=True)
    var = jnp.mean((z1 - mu) ** 2, axis=0, keepdims=True)
    z1 = (z1 - mu) * lax.rsqrt(var + 1e-5) * g1_ref[...] + be1_ref[...]
    z1 = jnp.maximum(z1, 0.0)
    z2 = jnp.dot(z1, w2t_ref[...], preferred_element_type=jnp.float32) + b2_ref[...]
    mu2 = jnp.mean(z2, axis=0, keepdims=True)
    var2 = jnp.mean((z2 - mu2) ** 2, axis=0, keepdims=True)
    z2 = (z2 - mu2) * lax.rsqrt(var2 + 1e-5) * g2_ref[...] + be2_ref[...]
    o_ref[0:N] = jnp.maximum(z2, 0.0)
    o_ref[N:] = jnp.zeros((NPADR, H), jnp.float32)


_mlp = pl.pallas_call(
    _mlp_body,
    out_shape=jax.ShapeDtypeStruct((N + NPADR, H), jnp.float32),
)


def _final_body(p_ref, h_ref, epsm1_ref, w1t_ref, b1_ref, g1_ref, be1_ref,
                w2t_ref, b2_ref, g2_ref, be2_ref, batch_ref,
                l1wt_ref, l1b_ref, l2wt_ref, l2b_ref, o_ref):
    h = h_ref[0:N]
    z = p_ref[0] + p_ref[1] + epsm1_ref[...] * h
    z1 = jnp.dot(z, w1t_ref[...], preferred_element_type=jnp.float32) + b1_ref[...]
    mu = jnp.mean(z1, axis=0, keepdims=True)
    var = jnp.mean((z1 - mu) ** 2, axis=0, keepdims=True)
    z1 = (z1 - mu) * lax.rsqrt(var + 1e-5) * g1_ref[...] + be1_ref[...]
    z1 = jnp.maximum(z1, 0.0)
    z2 = jnp.dot(z1, w2t_ref[...], preferred_element_type=jnp.float32) + b2_ref[...]
    mu2 = jnp.mean(z2, axis=0, keepdims=True)
    var2 = jnp.mean((z2 - mu2) ** 2, axis=0, keepdims=True)
    z2 = (z2 - mu2) * lax.rsqrt(var2 + 1e-5) * g2_ref[...] + be2_ref[...]
    hfin = jnp.maximum(z2, 0.0)

    iota = lax.broadcasted_iota(jnp.int32, (G, N), 0)
    onehot = jnp.where(batch_ref[...] == iota, 1.0, 0.0)
    sums = jnp.dot(onehot, hfin, preferred_element_type=jnp.float32)
    counts = jnp.sum(onehot, axis=1, keepdims=True)
    pooled = sums / jnp.maximum(counts, 1.0)
    zz = jnp.maximum(
        jnp.dot(pooled, l1wt_ref[...], preferred_element_type=jnp.float32)
        + l1b_ref[...], 0.0)
    o_ref[...] = (jnp.dot(zz, l2wt_ref[...], preferred_element_type=jnp.float32)
                  + l2b_ref[...])


_final = pl.pallas_call(
    _final_body,
    out_shape=jax.ShapeDtypeStruct((G, 1), jnp.float32),
)


def kernel(x, edge_index, batch, params):
    ppt = EPW - E // NW
    pad_src = jnp.full((NW, ppt), N, jnp.int32)
    pad_dst = jnp.broadcast_to(jnp.arange(ppt, dtype=jnp.int32), (NW, ppt))
    src_p = jnp.concatenate(
        [edge_index[0].reshape(NW, E // NW), pad_src], axis=1)
    dst_p = jnp.concatenate(
        [edge_index[1].reshape(NW, E // NW), pad_dst], axis=1)
    src = src_p.reshape(NW, NSUPER, KPS, CHK)
    dst = dst_p.reshape(NW, NSUPER, KPS, CHK)
    batch2d = batch.reshape(1, N)

    sc_agg = _make_sc_agg()
    h = jnp.concatenate([x, jnp.zeros((NPADR, D), jnp.float32)])
    out = None
    for l in range(NUM_LAYERS):
        p = sc_agg(h, src, dst)
        epsm1 = (params[f"eps_{l}"] - 1.0).reshape(1, 1)
        args = (
            p, h, epsm1,
            params[f"W1_{l}"].T, params[f"b1_{l}"].reshape(1, H),
            params[f"g1_{l}"].reshape(1, H), params[f"be1_{l}"].reshape(1, H),
            params[f"W2_{l}"].T, params[f"b2_{l}"].reshape(1, H),
            params[f"g2_{l}"].reshape(1, H), params[f"be2_{l}"].reshape(1, H),
        )
        if l < NUM_LAYERS - 1:
            h = _mlp(*args)
        else:
            out = _final(*args, batch2d,
                         params["lin1_W"].T, params["lin1_b"].reshape(1, H // 2),
                         params["lin2_W"].T, params["lin2_b"].reshape(1, 1))
    return out.squeeze(-1)


# speedup vs baseline: 1.7376x; 1.7376x over previous
import functools

import jax
import jax.numpy as jnp
from jax import lax
from jax.experimental import pallas as pl
from jax.experimental.pallas import tpu as pltpu
from jax.experimental.pallas import tpu_sc as plsc

N = 10000
E = 320000
D = 128
H = 128
G = 64
NUM_LAYERS = 3

NC = 2
NS = 16
NW = NC * NS
CHK = 128
KPS = 8
NSUPER = 10
NCHUNKT = NSUPER * KPS
EPW = NCHUNKT * CHK
SINK = 240
ACC_ROWS = N + SINK
RPT = 640
RPT_LAST = N - (NS - 1) * RPT


def _sc_agg_body(h_hbm, src_hbm, dst_hbm, out_hbm, src_v, didx0, didx1,
                 rows0, rows1, acc, gsem0, gsem1, isem0, isem1):
    c = lax.axis_index("c")
    s = lax.axis_index("s")
    w = c * NS + s

    pltpu.sync_copy(src_hbm.at[w], src_v)
    pltpu.sync_copy(dst_hbm.at[w, 0], didx0)

    @pl.when(s < NS - 1)
    def _():
        pltpu.sync_copy(h_hbm.at[pl.ds(s * RPT, RPT)],
                        acc.at[pl.ds(s * RPT, RPT)])

    @pl.when(s == NS - 1)
    def _():
        pltpu.sync_copy(h_hbm.at[pl.ds((NS - 1) * RPT, RPT_LAST)],
                        acc.at[pl.ds((NS - 1) * RPT, RPT_LAST)])

    plsc.subcore_barrier()

    rows = (rows0, rows1)
    gsems = (gsem0, gsem1)
    didx = (didx0, didx1)
    isems = (isem0, isem1)
    pltpu.async_copy(h_hbm.at[src_v.at[0]], rows0, gsem0)
    pltpu.async_copy(h_hbm.at[src_v.at[1]], rows1, gsem1)

    @pl.loop(0, NSUPER // 2)
    def _(jj):
        for jpar in range(2):
            j = jj * 2 + jpar
            jb = jpar

            @pl.when(j + 1 < NSUPER)
            def _():
                pltpu.async_copy(dst_hbm.at[w, j + 1], didx[1 - jb],
                                 isems[1 - jb])

            @pl.when(j > 0)
            def _():
                pltpu.make_async_copy(dst_hbm.at[w, j], didx[jb],
                                      isems[jb]).wait()

            for k in range(KPS):
                i = j * KPS + k
                b = k % 2
                pltpu.make_async_copy(h_hbm.at[src_v.at[i]], rows[b],
                                      gsems[b]).wait()
                pltpu.sync_copy(rows[b], acc.at[didx[jb].at[k]], add=True)

                @pl.when(i + 2 < NCHUNKT)
                def _():
                    pltpu.async_copy(h_hbm.at[src_v.at[i + 2]], rows[b],
                                     gsems[b])

    plsc.subcore_barrier()

    @pl.when(s < NS - 1)
    def _():
        pltpu.sync_copy(acc.at[pl.ds(s * RPT, RPT)],
                        out_hbm.at[c, pl.ds(s * RPT, RPT)])

    @pl.when(s == NS - 1)
    def _():
        pltpu.sync_copy(acc.at[pl.ds((NS - 1) * RPT, RPT_LAST)],
                        out_hbm.at[c, pl.ds((NS - 1) * RPT, RPT_LAST)])


@functools.cache
def _make_sc_agg():
    mesh = plsc.VectorSubcoreMesh(
        core_axis_name="c", subcore_axis_name="s", num_cores=NC, num_subcores=NS
    )
    return pl.kernel(
        _sc_agg_body,
        out_type=jax.ShapeDtypeStruct((NC, N, D), jnp.float32),
        mesh=mesh,
        scratch_types=[
            pltpu.VMEM((NCHUNKT, CHK), jnp.int32),
            pltpu.VMEM((KPS, CHK), jnp.int32),
            pltpu.VMEM((KPS, CHK), jnp.int32),
            pltpu.VMEM((CHK, D), jnp.float32),
            pltpu.VMEM((CHK, D), jnp.float32),
            pltpu.VMEM_SHARED((ACC_ROWS, D), jnp.float32),
            pltpu.SemaphoreType.DMA,
            pltpu.SemaphoreType.DMA,
            pltpu.SemaphoreType.DMA,
            pltpu.SemaphoreType.DMA,
        ],
    )


def _mlp_body(p_ref, h_ref, epsm1_ref, w1t_ref, b1_ref, g1_ref, be1_ref,
              w2t_ref, b2_ref, g2_ref, be2_ref, o_ref):
    h = h_ref[...]
    z = p_ref[0] + p_ref[1] + epsm1_ref[...] * h
    z1 = jnp.dot(z, w1t_ref[...], preferred_element_type=jnp.float32) + b1_ref[...]
    mu = jnp.mean(z1, axis=0, keepdims=True)
    var = jnp.mean((z1 - mu) ** 2, axis=0, keepdims=True)
    z1 = (z1 - mu) * lax.rsqrt(var + 1e-5) * g1_ref[...] + be1_ref[...]
    z1 = jnp.maximum(z1, 0.0)
    z2 = jnp.dot(z1, w2t_ref[...], preferred_element_type=jnp.float32) + b2_ref[...]
    mu2 = jnp.mean(z2, axis=0, keepdims=True)
    var2 = jnp.mean((z2 - mu2) ** 2, axis=0, keepdims=True)
    z2 = (z2 - mu2) * lax.rsqrt(var2 + 1e-5) * g2_ref[...] + be2_ref[...]
    o_ref[...] = jnp.maximum(z2, 0.0)


_mlp = pl.pallas_call(
    _mlp_body,
    out_shape=jax.ShapeDtypeStruct((N, H), jnp.float32),
)


def _final_body(p_ref, h_ref, epsm1_ref, w1t_ref, b1_ref, g1_ref, be1_ref,
                w2t_ref, b2_ref, g2_ref, be2_ref, batch_ref,
                l1wt_ref, l1b_ref, l2wt_ref, l2b_ref, o_ref):
    h = h_ref[...]
    z = p_ref[0] + p_ref[1] + epsm1_ref[...] * h
    z1 = jnp.dot(z, w1t_ref[...], preferred_element_type=jnp.float32) + b1_ref[...]
    mu = jnp.mean(z1, axis=0, keepdims=True)
    var = jnp.mean((z1 - mu) ** 2, axis=0, keepdims=True)
    z1 = (z1 - mu) * lax.rsqrt(var + 1e-5) * g1_ref[...] + be1_ref[...]
    z1 = jnp.maximum(z1, 0.0)
    z2 = jnp.dot(z1, w2t_ref[...], preferred_element_type=jnp.float32) + b2_ref[...]
    mu2 = jnp.mean(z2, axis=0, keepdims=True)
    var2 = jnp.mean((z2 - mu2) ** 2, axis=0, keepdims=True)
    z2 = (z2 - mu2) * lax.rsqrt(var2 + 1e-5) * g2_ref[...] + be2_ref[...]
    hfin = jnp.maximum(z2, 0.0)

    iota = lax.broadcasted_iota(jnp.int32, (G, N), 0)
    onehot = jnp.where(batch_ref[...] == iota, 1.0, 0.0)
    sums = jnp.dot(onehot, hfin, preferred_element_type=jnp.float32)
    counts = jnp.sum(onehot, axis=1, keepdims=True)
    pooled = sums / jnp.maximum(counts, 1.0)
    zz = jnp.maximum(
        jnp.dot(pooled, l1wt_ref[...], preferred_element_type=jnp.float32)
        + l1b_ref[...], 0.0)
    o_ref[...] = (jnp.dot(zz, l2wt_ref[...], preferred_element_type=jnp.float32)
                  + l2b_ref[...])


_final = pl.pallas_call(
    _final_body,
    out_shape=jax.ShapeDtypeStruct((G, 1), jnp.float32),
)


def kernel(x, edge_index, batch, params):
    ppt = EPW - E // NW
    pad_src = jnp.broadcast_to(
        (jnp.arange(ppt, dtype=jnp.int32) * 41) % N, (NW, ppt))
    pad_dst = jnp.broadcast_to(
        N + jnp.arange(ppt, dtype=jnp.int32), (NW, ppt))
    src_p = jnp.concatenate(
        [edge_index[0].reshape(NW, E // NW), pad_src], axis=1)
    dst_p = jnp.concatenate(
        [edge_index[1].reshape(NW, E // NW), pad_dst], axis=1)
    src = src_p.reshape(NW, NCHUNKT, CHK)
    dst = dst_p.reshape(NW, NSUPER, KPS, CHK)
    batch2d = batch.reshape(1, N)

    sc_agg = _make_sc_agg()
    h = x
    out = None
    for l in range(NUM_LAYERS):
        p = sc_agg(h, src, dst)
        epsm1 = (params[f"eps_{l}"] - 1.0).reshape(1, 1)
        args = (
            p, h, epsm1,
            params[f"W1_{l}"].T, params[f"b1_{l}"].reshape(1, H),
            params[f"g1_{l}"].reshape(1, H), params[f"be1_{l}"].reshape(1, H),
            params[f"W2_{l}"].T, params[f"b2_{l}"].reshape(1, H),
            params[f"g2_{l}"].reshape(1, H), params[f"be2_{l}"].reshape(1, H),
        )
        if l < NUM_LAYERS - 1:
            h = _mlp(*args)
        else:
            out = _final(*args, batch2d,
                         params["lin1_W"].T, params["lin1_b"].reshape(1, H // 2),
                         params["lin2_W"].T, params["lin2_b"].reshape(1, 1))
    return out.squeeze(-1)
